# baseline (speedup 1.0000x reference)
import functools

import jax
import jax.numpy as jnp
from jax import lax
from jax.experimental import pallas as pl
from jax.experimental.pallas import tpu as pltpu
from jax.experimental.pallas import tpu_sc as plsc

N = 10000
E = 320000
NG = 64
D = 128
BN_EPS = 1e-5

NC = 2
NS = 16
NW = NC * NS
CH = 64
EPAD = 327680
NCH = EPAD // (NW * CH)
NPAD = 10240
RPS = NPAD // NS
NB = 4
LK = 2


def _sc_aggregate(h, src3, dst3, zeros):
    mesh = plsc.VectorSubcoreMesh(
        core_axis_name="c", subcore_axis_name="s", num_cores=NC, num_subcores=NS
    )

    @functools.partial(
        pl.kernel,
        out_type=jax.ShapeDtypeStruct((NC, NPAD, D), jnp.float32),
        mesh=mesh,
        scratch_types=[
            pltpu.VMEM((NCH * CH,), jnp.int32),
            pltpu.VMEM((NB, 1, CH), jnp.int32),
            pltpu.VMEM((NB, CH, D), jnp.float32),
            pltpu.VMEM_SHARED((NPAD, D), jnp.float32),
        ]
        + [pltpu.SemaphoreType.DMA] * (3 * NB),
    )
    def agg_kernel(h_hbm, src_hbm, dst_hbm, z_hbm, out_hbm,
                   sidx, didx, rows, acc, *sems):
        gsem = sems[:NB]
        ssem = sems[NB:2 * NB]
        dsem = sems[2 * NB:]
        c = lax.axis_index("c")
        s = lax.axis_index("s")
        w = c * NS + s

        pltpu.sync_copy(src_hbm.at[pl.ds(w * (NCH * CH), NCH * CH)], sidx)
        pltpu.sync_copy(z_hbm.at[pl.ds(s * RPS, RPS)],
                        acc.at[pl.ds(s * RPS, RPS)])
        plsc.subcore_barrier()

        def fire_didx(j, b):
            pltpu.async_copy(dst_hbm.at[w * NCH + j], didx.at[b], dsem[b])

        def wait_didx(b):
            pltpu.make_async_copy(dst_hbm.at[0], didx.at[b], dsem[b]).wait()

        def fire_gather(j, b):
            off = pl.multiple_of(j * CH, CH)
            pltpu.async_copy(h_hbm.at[sidx.at[pl.ds(off, CH)]], rows.at[b],
                             gsem[b])

        def wait_gather(b):
            pltpu.make_async_copy(h_hbm.at[sidx.at[pl.ds(0, CH)]], rows.at[b],
                                  gsem[b]).wait()

        def fire_scatter(j, b):
            pltpu.async_copy(rows.at[b], acc.at[didx.at[b].at[0]], ssem[b],
                             add=True)

        def wait_scatter(b):
            pltpu.make_async_copy(rows.at[b], acc.at[didx.at[0].at[0]],
                                  ssem[b]).wait()

        for j in range(LK):
            fire_didx(j, j % NB)
            fire_gather(j, j % NB)
        for j in range(LK):
            fire_didx(j + LK, (j + LK) % NB)
            fire_gather(j + LK, (j + LK) % NB)
            wait_gather(j % NB)
            wait_didx(j % NB)
            fire_scatter(j, j % NB)

        @pl.loop(LK, NCH - LK, step=NB)
        def _(g):
            for u in range(NB):
                j = g + u
                b = (LK + u) % NB
                bg = (LK + u + LK) % NB
                wait_scatter(bg)
                fire_didx(j + LK, bg)
                fire_gather(j + LK, bg)
                wait_gather(b)
                wait_didx(b)
                fire_scatter(j, b)

        for j in range(NCH - LK, NCH):
            wait_gather(j % NB)
            wait_didx(j % NB)
            fire_scatter(j, j % NB)
        for j in range(NCH - NB, NCH):
            wait_scatter(j % NB)

        plsc.subcore_barrier()
        pltpu.sync_copy(acc.at[pl.ds(s * RPS, RPS)],
                        out_hbm.at[c].at[pl.ds(s * RPS, RPS)])

    return agg_kernel(h, src3, dst3, zeros)


def _tc_layer_body(h_ref, p_ref, w1_ref, b1_ref, w2_ref, b2_ref,
                   g_ref, be_ref, o_ref):
    hs = h_ref[...] + p_ref[0, :N, :] + p_ref[1, :N, :]
    a = jnp.maximum(
        jnp.dot(hs, w1_ref[...], preferred_element_type=jnp.float32)
        + b1_ref[...], 0.0)
    h2 = (jnp.dot(a, w2_ref[...], preferred_element_type=jnp.float32)
          + b2_ref[...])
    m = jnp.mean(h2, axis=0, keepdims=True)
    v = jnp.mean((h2 - m) * (h2 - m), axis=0, keepdims=True)
    o_ref[...] = jnp.maximum(
        (h2 - m) * jax.lax.rsqrt(v + BN_EPS) * g_ref[...] + be_ref[...], 0.0)


def _tc_layer(h, p, W1, b1, W2, b2, g, be):
    return pl.pallas_call(
        _tc_layer_body,
        out_shape=jax.ShapeDtypeStruct((N, D), jnp.float32),
    )(h, p, W1, b1, W2, b2, g, be)


def _tc_layer_pool_body(h_ref, p_ref, w1_ref, b1_ref, w2_ref, b2_ref,
                        g_ref, be_ref, batch_ref, o_ref):
    hs = h_ref[...] + p_ref[0, :N, :] + p_ref[1, :N, :]
    a = jnp.maximum(
        jnp.dot(hs, w1_ref[...], preferred_element_type=jnp.float32)
        + b1_ref[...], 0.0)
    h2 = (jnp.dot(a, w2_ref[...], preferred_element_type=jnp.float32)
          + b2_ref[...])
    m = jnp.mean(h2, axis=0, keepdims=True)
    v = jnp.mean((h2 - m) * (h2 - m), axis=0, keepdims=True)
    hf = jnp.maximum(
        (h2 - m) * jax.lax.rsqrt(v + BN_EPS) * g_ref[...] + be_ref[...], 0.0)
    gids = lax.broadcasted_iota(jnp.int32, (N, NG), 1)
    onehot = (batch_ref[...] == gids).astype(jnp.float32)
    sums = lax.dot_general(onehot, hf, (((0,), (0,)), ((), ())),
                           preferred_element_type=jnp.float32)
    cnt = lax.dot_general(onehot, jnp.ones((N, 1), jnp.float32),
                          (((0,), (0,)), ((), ())),
                          preferred_element_type=jnp.float32)
    o_ref[...] = sums / jnp.clip(cnt, 1.0, None)


def _tc_layer_pool(h, p, W1, b1, W2, b2, g, be, batch):
    return pl.pallas_call(
        _tc_layer_pool_body,
        out_shape=jax.ShapeDtypeStruct((NG, D), jnp.float32),
    )(h, p, W1, b1, W2, b2, g, be, batch)


def kernel(x, edge_index, batch,
           W1_0, b1_0, W2_0, b2_0, g_0, be_0,
           W1_1, b1_1, W2_1, b2_1, g_1, be_1,
           W1_2, b1_2, W2_2, b2_2, g_2, be_2):
    pad = EPAD - E
    src1 = jnp.concatenate([edge_index[0], jnp.zeros((pad,), jnp.int32)])
    dst3 = jnp.concatenate(
        [edge_index[1], jnp.full((pad,), N, jnp.int32)]).reshape(NW * NCH, 1, CH)
    zeros = jnp.zeros((NPAD, D), jnp.float32)
    batch2d = batch.reshape(N, 1)
    params = [(W1_0, b1_0, W2_0, b2_0, g_0, be_0),
              (W1_1, b1_1, W2_1, b2_1, g_1, be_1),
              (W1_2, b1_2, W2_2, b2_2, g_2, be_2)]

    h = x
    for i, (W1, b1, W2, b2, g, be) in enumerate(params):
        p = _sc_aggregate(h, src1, dst3, zeros)
        b1r = b1.reshape(1, D)
        b2r = b2.reshape(1, D)
        gr = g.reshape(1, D)
        ber = be.reshape(1, D)
        if i < 2:
            h = _tc_layer(h, p, W1, b1r, W2, b2r, gr, ber)
        else:
            h = _tc_layer_pool(h, p, W1, b1r, W2, b2r, gr, ber, batch2d)
    return h
